# Initial kernel scaffold; baseline (speedup 1.0000x reference)
import jax
import jax.numpy as jnp
from jax.experimental import pallas as pl


def kernel(cards, rank_embed, suit_embed):
    raise NotImplementedError("write your pallas kernel here")


# speedup vs baseline: 350.7962x; 350.7962x over previous
import functools

import jax
import jax.numpy as jnp
from jax import lax
from jax.experimental import pallas as pl
from jax.experimental.pallas import tpu as pltpu
from jax.experimental.pallas import tpu_sc as plsc

_LANES = 16
_NC = 2
_NS = 16
_BINS = 52
_UNROLL = 8


@functools.partial(jax.jit, static_argnums=(0,))
def _histogram_embed(n, cards, ew):
    nw = _NC * _NS
    chunk = n // nw
    vecs = chunk // _LANES
    assert chunk % (_UNROLL * _LANES) == 0 and n == chunk * nw

    mesh = plsc.VectorSubcoreMesh(
        core_axis_name="c", subcore_axis_name="s",
        num_cores=_NC, num_subcores=_NS)

    @functools.partial(
        pl.kernel,
        out_type=jax.ShapeDtypeStruct((nw, _LANES), jnp.float32),
        mesh=mesh,
        compiler_params=pltpu.CompilerParams(needs_layout_passes=False),
        scratch_types=[
            pltpu.VMEM((chunk,), jnp.int32),
            pltpu.VMEM((_BINS * _LANES,), jnp.float32),
            pltpu.VMEM((_BINS, _LANES), jnp.float32),
            pltpu.VMEM((_LANES,), jnp.float32),
        ],
    )
    def card_embed(cards_hbm, ew_hbm, out_hbm,
                   cards_v, hist_v, ew_v, acc_v):
        cid = lax.axis_index("c")
        sid = lax.axis_index("s")
        wid = sid * _NC + cid
        base = wid * chunk

        pltpu.sync_copy(cards_hbm.at[pl.ds(base, chunk)], cards_v)
        pltpu.sync_copy(ew_hbm, ew_v)

        zeros = jnp.zeros((_LANES,), jnp.float32)
        for b in range(_BINS):
            hist_v[pl.ds(b * _LANES, _LANES)] = zeros

        lanes = lax.broadcasted_iota(jnp.int32, (_LANES,), 0)
        ones = jnp.ones((_LANES,), jnp.float32)

        def hbody(i, carry):
            b0 = i * (_UNROLL * _LANES)
            for u in range(_UNROLL):
                c = cards_v[pl.ds(b0 + u * _LANES, _LANES)]
                plsc.addupdate_scatter(hist_v, [c * _LANES + lanes], ones)
            return carry

        lax.fori_loop(0, vecs // _UNROLL, hbody, 0)

        inv_n = jnp.float32(1.0 / n)

        def ebody(b, acc):
            cnt = jnp.sum(hist_v[pl.ds(b * _LANES, _LANES)])
            return acc + cnt * ew_v[b]

        acc = lax.fori_loop(0, _BINS, ebody, zeros)
        acc_v[...] = acc * inv_n

        pltpu.sync_copy(acc_v, out_hbm.at[wid])

    return card_embed(cards, ew)


def kernel(cards, rank_embed, suit_embed):
    n = cards.shape[0]
    cards = cards.astype(jnp.int32)
    ew = jnp.concatenate([
        jnp.tile(rank_embed, (4, 1)),
        jnp.repeat(suit_embed, 13, axis=0),
        jnp.zeros((_BINS, 4), jnp.float32),
    ], axis=1)
    parts = _histogram_embed(n, cards, ew)
    return jnp.sum(parts, axis=0)[:12]


# speedup vs baseline: 734.7064x; 2.0944x over previous
import functools

import jax
import jax.numpy as jnp
from jax import lax
from jax.experimental import pallas as pl
from jax.experimental.pallas import tpu as pltpu
from jax.experimental.pallas import tpu_sc as plsc

_LANES = 16
_NC = 2
_NS = 16
_BINS = 52
_UNROLL = 8


@functools.partial(jax.jit, static_argnums=(0,))
def _histogram_embed(n, cards, ew):
    nw = _NC * _NS
    chunk = n // nw
    vecs = chunk // _LANES
    assert chunk % (_UNROLL * _LANES) == 0 and n == chunk * nw

    mesh = plsc.VectorSubcoreMesh(
        core_axis_name="c", subcore_axis_name="s",
        num_cores=_NC, num_subcores=_NS)

    @functools.partial(
        pl.kernel,
        out_type=jax.ShapeDtypeStruct((nw, _LANES), jnp.float32),
        mesh=mesh,
        compiler_params=pltpu.CompilerParams(needs_layout_passes=False),
        scratch_types=[
            pltpu.VMEM((chunk,), jnp.int32),
            pltpu.VMEM((_BINS * _LANES,), jnp.float32),
            pltpu.VMEM((_BINS, _LANES), jnp.float32),
            pltpu.VMEM((_LANES,), jnp.float32),
        ],
    )
    def card_embed(cards_hbm, ew_hbm, out_hbm,
                   cards_v, hist_v, ew_v, acc_v):
        cid = lax.axis_index("c")
        sid = lax.axis_index("s")
        wid = sid * _NC + cid
        base = wid * chunk

        pltpu.sync_copy(cards_hbm.at[pl.ds(base, chunk)], cards_v)
        pltpu.sync_copy(ew_hbm, ew_v)

        zeros = jnp.zeros((_LANES,), jnp.float32)
        for b in range(_BINS):
            hist_v[pl.ds(b * _LANES, _LANES)] = zeros

        lanes = lax.broadcasted_iota(jnp.int32, (_LANES,), 0)
        ones = jnp.ones((_LANES,), jnp.float32)

        @plsc.parallel_loop(0, vecs, step=1, unroll=_UNROLL)
        def _hist(i):
            c = cards_v[pl.ds(i * _LANES, _LANES)]
            plsc.addupdate_scatter(hist_v, [c * _LANES + lanes], ones)

        inv_n = jnp.float32(1.0 / n)

        def ebody(b, acc):
            cnt = jnp.sum(hist_v[pl.ds(b * _LANES, _LANES)])
            return acc + cnt * ew_v[b]

        acc = lax.fori_loop(0, _BINS, ebody, zeros)
        acc_v[...] = acc * inv_n

        pltpu.sync_copy(acc_v, out_hbm.at[wid])

    return card_embed(cards, ew)


def kernel(cards, rank_embed, suit_embed):
    n = cards.shape[0]
    cards = cards.astype(jnp.int32)
    ew = jnp.concatenate([
        jnp.tile(rank_embed, (4, 1)),
        jnp.repeat(suit_embed, 13, axis=0),
        jnp.zeros((_BINS, 4), jnp.float32),
    ], axis=1)
    parts = _histogram_embed(n, cards, ew)
    return jnp.sum(parts, axis=0)[:12]


# speedup vs baseline: 753.6394x; 1.0258x over previous
import functools

import jax
import jax.numpy as jnp
from jax import lax
from jax.experimental import pallas as pl
from jax.experimental.pallas import tpu as pltpu
from jax.experimental.pallas import tpu_sc as plsc

_LANES = 16
_NC = 2
_NS = 16
_BINS = 52
_UNROLL = 8
_NCHUNK = 8


@functools.partial(jax.jit, static_argnums=(0,))
def _histogram_embed(n, cards, rank_embed, suit_embed):
    nw = _NC * _NS
    chunk = n // nw
    csub = chunk // _NCHUNK
    vec_sub = csub // _LANES
    assert n == chunk * nw and csub % (_UNROLL * _LANES) == 0

    mesh = plsc.VectorSubcoreMesh(
        core_axis_name="c", subcore_axis_name="s",
        num_cores=_NC, num_subcores=_NS)

    @functools.partial(
        pl.kernel,
        out_type=jax.ShapeDtypeStruct((nw, _LANES), jnp.float32),
        mesh=mesh,
        compiler_params=pltpu.CompilerParams(needs_layout_passes=False),
        scratch_types=[
            pltpu.VMEM((csub,), jnp.int32),
            pltpu.VMEM((csub,), jnp.int32),
            pltpu.VMEM((_BINS * _LANES,), jnp.float32),
            pltpu.VMEM((13, 8), jnp.float32),
            pltpu.VMEM((4, 4), jnp.float32),
            pltpu.VMEM((_LANES,), jnp.float32),
            pltpu.SemaphoreType.DMA,
            pltpu.SemaphoreType.DMA,
        ],
    )
    def card_embed(cards_hbm, re_hbm, se_hbm, out_hbm,
                   buf0, buf1, hist_v, re_v, se_v, acc_v, sem0, sem1):
        cid = lax.axis_index("c")
        sid = lax.axis_index("s")
        wid = sid * _NC + cid
        base = wid * chunk

        bufs, sems = (buf0, buf1), (sem0, sem1)
        descs = [None, None]
        descs[0] = pltpu.async_copy(
            cards_hbm.at[pl.ds(base, csub)], buf0, sem0)

        pltpu.sync_copy(re_hbm, re_v)
        pltpu.sync_copy(se_hbm, se_v)
        zeros = jnp.zeros((_LANES,), jnp.float32)
        for b in range(_BINS):
            hist_v[pl.ds(b * _LANES, _LANES)] = zeros

        lanes = lax.broadcasted_iota(jnp.int32, (_LANES,), 0)
        ones = jnp.ones((_LANES,), jnp.float32)

        for k in range(_NCHUNK):
            descs[k % 2].wait()
            if k + 1 < _NCHUNK:
                descs[(k + 1) % 2] = pltpu.async_copy(
                    cards_hbm.at[pl.ds(base + (k + 1) * csub, csub)],
                    bufs[(k + 1) % 2], sems[(k + 1) % 2])
            bv = bufs[k % 2]

            @plsc.parallel_loop(0, vec_sub, step=1, unroll=_UNROLL)
            def _hist(i, bv=bv):
                c = bv[pl.ds(i * _LANES, _LANES)]
                plsc.addupdate_scatter(hist_v, [c * _LANES + lanes], ones)

        mask_r = lanes < 8
        mask_s = jnp.logical_and(lanes >= 8, lanes < 12)
        lanes7 = jnp.bitwise_and(lanes, 7)
        lanes3 = jnp.bitwise_and(lanes, 3)

        def ebody(b, acc):
            cnt = jnp.sum(hist_v[pl.ds(b * _LANES, _LANES)])
            r = b % 13
            s = b // 13
            g_r = plsc.load_gather(re_v, [jnp.full((_LANES,), r, jnp.int32), lanes7])
            g_s = plsc.load_gather(se_v, [jnp.full((_LANES,), s, jnp.int32), lanes3])
            w = jnp.where(mask_r, g_r, zeros) + jnp.where(mask_s, g_s, zeros)
            return acc + cnt * w

        acc = lax.fori_loop(0, _BINS, ebody, zeros)
        acc_v[...] = acc * jnp.float32(1.0 / n)
        pltpu.sync_copy(acc_v, out_hbm.at[wid])

    return card_embed(cards, rank_embed, suit_embed)


def kernel(cards, rank_embed, suit_embed):
    n = cards.shape[0]
    cards = cards.astype(jnp.int32)
    parts = _histogram_embed(n, cards, rank_embed, suit_embed)
    return jnp.sum(parts, axis=0)[:12]


# speedup vs baseline: 781.6788x; 1.0372x over previous
import functools

import jax
import jax.numpy as jnp
from jax import lax
from jax.experimental import pallas as pl
from jax.experimental.pallas import tpu as pltpu
from jax.experimental.pallas import tpu_sc as plsc

_LANES = 16
_NC = 2
_NS = 16
_BINS = 52
_UNROLL = 8
_NCHUNK = 8


@functools.partial(jax.jit, static_argnums=(0,))
def _histogram_embed(n, cards, rank_embed, suit_embed):
    nw = _NC * _NS
    chunk = n // nw
    csub = chunk // _NCHUNK
    vec_sub = csub // _LANES
    assert n == chunk * nw and csub % (_UNROLL * _LANES) == 0

    mesh = plsc.VectorSubcoreMesh(
        core_axis_name="c", subcore_axis_name="s",
        num_cores=_NC, num_subcores=_NS)

    @functools.partial(
        pl.kernel,
        out_type=jax.ShapeDtypeStruct((nw, _LANES), jnp.float32),
        mesh=mesh,
        compiler_params=pltpu.CompilerParams(needs_layout_passes=False),
        scratch_types=[
            pltpu.VMEM((csub,), jnp.int32),
            pltpu.VMEM((csub,), jnp.int32),
            pltpu.VMEM((_BINS * _LANES,), jnp.float32),
            pltpu.VMEM((13, 8), jnp.float32),
            pltpu.VMEM((4, 4), jnp.float32),
            pltpu.VMEM((_LANES,), jnp.float32),
            pltpu.SemaphoreType.DMA,
            pltpu.SemaphoreType.DMA,
        ],
    )
    def card_embed(cards_hbm, re_hbm, se_hbm, out_hbm,
                   buf0, buf1, hist_v, re_v, se_v, acc_v, sem0, sem1):
        cid = lax.axis_index("c")
        sid = lax.axis_index("s")
        wid = sid * _NC + cid
        base = wid * chunk

        pltpu.async_copy(cards_hbm.at[pl.ds(base, csub)], buf0, sem0)
        pltpu.async_copy(cards_hbm.at[pl.ds(base + csub, csub)], buf1, sem1)

        pltpu.sync_copy(re_hbm, re_v)
        pltpu.sync_copy(se_hbm, se_v)
        zeros = jnp.zeros((_LANES,), jnp.float32)

        def zbody(b, carry):
            hist_v[pl.ds(b * _LANES, _LANES)] = zeros
            return carry

        lax.fori_loop(0, _BINS, zbody, 0)

        lanes = lax.broadcasted_iota(jnp.int32, (_LANES,), 0)
        ones = jnp.ones((_LANES,), jnp.float32)
        npair = _NCHUNK // 2

        def pair_body(j, carry):
            for b, (bv, sem) in enumerate(((buf0, sem0), (buf1, sem1))):
                pltpu.make_async_copy(
                    cards_hbm.at[pl.ds(base, csub)], bv, sem).wait()

                @plsc.parallel_loop(0, vec_sub, step=1, unroll=_UNROLL)
                def _hist(i, bv=bv):
                    c = bv[pl.ds(i * _LANES, _LANES)]
                    plsc.addupdate_scatter(hist_v, [c * _LANES + lanes], ones)

                @pl.when(j + 1 < npair)
                def _():
                    nxt = base + (2 * (j + 1) + b) * csub
                    pltpu.async_copy(cards_hbm.at[pl.ds(nxt, csub)], bv, sem)
            return carry

        lax.fori_loop(0, npair, pair_body, 0)

        mask_r = lanes < 8
        mask_s = jnp.logical_and(lanes >= 8, lanes < 12)
        lanes7 = jnp.bitwise_and(lanes, 7)
        lanes3 = jnp.bitwise_and(lanes, 3)

        def ebody(b, acc):
            cnt = jnp.sum(hist_v[pl.ds(b * _LANES, _LANES)])
            r = b % 13
            s = b // 13
            g_r = plsc.load_gather(re_v, [jnp.full((_LANES,), r, jnp.int32), lanes7])
            g_s = plsc.load_gather(se_v, [jnp.full((_LANES,), s, jnp.int32), lanes3])
            w = jnp.where(mask_r, g_r, zeros) + jnp.where(mask_s, g_s, zeros)
            return acc + cnt * w

        acc = lax.fori_loop(0, _BINS, ebody, zeros)
        acc_v[...] = acc * jnp.float32(1.0 / n)
        pltpu.sync_copy(acc_v, out_hbm.at[wid])

    return card_embed(cards, rank_embed, suit_embed)


def kernel(cards, rank_embed, suit_embed):
    n = cards.shape[0]
    cards = cards.astype(jnp.int32)
    parts = _histogram_embed(n, cards, rank_embed, suit_embed)
    return jnp.sum(parts, axis=0)[:12]
